# Initial kernel scaffold; baseline (speedup 1.0000x reference)
import jax
import jax.numpy as jnp
from jax.experimental import pallas as pl


def kernel(h, edge_index, W, beta):
    raise NotImplementedError("write your pallas kernel here")


# speedup vs baseline: 3.9187x; 3.9187x over previous
import functools

import jax
import jax.numpy as jnp
from jax import lax
from jax.experimental import pallas as pl
from jax.experimental.pallas import tpu as pltpu
from jax.experimental.pallas import tpu_sc as plsc

D = 128
BLK = 128
NW = 32
NSUB = 16


def _prep_body(beta_ref, h_ref, w_ref, z_ref, zh_ref):
    z = lax.dot_general(h_ref[...], w_ref[...], (((1,), (1,)), ((), ())),
                        preferred_element_type=jnp.float32)
    z_ref[...] = z
    nrm = jnp.sqrt(jnp.sum(z * z, axis=1, keepdims=True))
    scale = jnp.sqrt(beta_ref[0, 0]) / jnp.maximum(nrm, 1e-6)
    zh_ref[...] = z * scale


def _prep(h, W, beta):
    n = h.shape[0]
    rb = 1000
    grid = n // rb
    return pl.pallas_call(
        _prep_body,
        grid=(grid,),
        in_specs=[
            pl.BlockSpec((1, 1), lambda i: (0, 0)),
            pl.BlockSpec((rb, D), lambda i: (i, 0)),
            pl.BlockSpec((D, D), lambda i: (0, 0)),
        ],
        out_specs=[
            pl.BlockSpec((rb, D), lambda i: (i, 0)),
            pl.BlockSpec((rb, D), lambda i: (i, 0)),
        ],
        out_shape=[
            jax.ShapeDtypeStruct((n, D), jnp.float32),
            jax.ShapeDtypeStruct((n, D), jnp.float32),
        ],
    )(beta.reshape(1, 1), h, W)


def _make_k1(e, npad):
    nblk = e // BLK
    iters = pl.cdiv(nblk, NW)
    sl = npad // NSUB
    mesh = plsc.VectorSubcoreMesh(core_axis_name="c", subcore_axis_name="s", num_cores=2, num_subcores=16)

    @functools.partial(
        pl.kernel,
        mesh=mesh,
        compiler_params=pltpu.CompilerParams(needs_layout_passes=False),
        out_type=(
            jax.ShapeDtypeStruct((e,), jnp.float32),
            jax.ShapeDtypeStruct((npad,), jnp.float32),
            jax.ShapeDtypeStruct((npad,), jnp.float32),
        ),
        scratch_types=[
            pltpu.VMEM((BLK,), jnp.int32),
            pltpu.VMEM((BLK,), jnp.int32),
            pltpu.VMEM((BLK, D), jnp.float32),
            pltpu.VMEM((BLK, D), jnp.float32),
            pltpu.VMEM((BLK,), jnp.float32),
            pltpu.VMEM((sl,), jnp.float32),
            pltpu.VMEM_SHARED((npad,), jnp.float32),
            pltpu.SemaphoreType.DMA,
            pltpu.SemaphoreType.DMA,
        ],
    )
    def k1(zh_hbm, src_hbm, dst_hbm, w_hbm, s0_hbm, s1_hbm,
           src_v, dst_v, zs_v, zd_v, w_v, sbuf_v, s_sh, sem1, sem2):
        cid = lax.axis_index("c")
        sid = lax.axis_index("s")
        wid = cid * NSUB + sid

        zero16 = jnp.zeros((16,), jnp.float32)
        def zbody(i, c):
            sbuf_v[pl.ds(i * 16, 16)] = zero16
            return c
        lax.fori_loop(0, sl // 16, zbody, 0)
        pltpu.sync_copy(sbuf_v, s_sh.at[pl.ds(sid * sl, sl)])
        plsc.subcore_barrier()

        def block_body(i, c):
            b = wid + i * NW

            @pl.when(b < nblk)
            def _():
                base = b * BLK
                pltpu.sync_copy(src_hbm.at[pl.ds(base, BLK)], src_v)
                pltpu.sync_copy(dst_hbm.at[pl.ds(base, BLK)], dst_v)
                cp1 = pltpu.async_copy(zh_hbm.at[src_v], zs_v, sem1)
                cp2 = pltpu.async_copy(zh_hbm.at[dst_v], zd_v, sem2)
                cp1.wait()
                cp2.wait()
                for g in range(BLK // 16):
                    rows = lax.iota(jnp.int32, 16) + g * 16

                    def dbody(j, acc):
                        for k in range(8):
                            col = jnp.full((16,), j * 8 + k, jnp.int32)
                            a = plsc.load_gather(zs_v, [rows, col])
                            bv = plsc.load_gather(zd_v, [rows, col])
                            acc = acc + a * bv
                        return acc

                    acc = lax.fori_loop(0, D // 8, dbody,
                                        jnp.zeros((16,), jnp.float32))
                    w_v[pl.ds(g * 16, 16)] = jnp.exp(acc)
                pltpu.sync_copy(w_v, w_hbm.at[pl.ds(base, BLK)])
                pltpu.sync_copy(w_v, s_sh.at[dst_v], add=True)
            return c

        lax.fori_loop(0, iters, block_body, 0)
        plsc.subcore_barrier()

        pltpu.sync_copy(s_sh.at[pl.ds(sid * sl, sl)], sbuf_v)

        @pl.when(cid == 0)
        def _():
            pltpu.sync_copy(sbuf_v, s0_hbm.at[pl.ds(sid * sl, sl)])

        @pl.when(cid == 1)
        def _():
            pltpu.sync_copy(sbuf_v, s1_hbm.at[pl.ds(sid * sl, sl)])

    return k1


def _make_k2(e, npad):
    nblk = e // BLK
    iters = pl.cdiv(nblk, NW)
    sl = npad // NSUB
    rows_per_copy = 128
    mesh = plsc.VectorSubcoreMesh(core_axis_name="c", subcore_axis_name="s", num_cores=2, num_subcores=16)

    @functools.partial(
        pl.kernel,
        mesh=mesh,
        compiler_params=pltpu.CompilerParams(needs_layout_passes=False),
        out_type=(
            jax.ShapeDtypeStruct((npad, D), jnp.float32),
            jax.ShapeDtypeStruct((npad, D), jnp.float32),
        ),
        scratch_types=[
            pltpu.VMEM((BLK,), jnp.int32),
            pltpu.VMEM((BLK,), jnp.int32),
            pltpu.VMEM((BLK,), jnp.float32),
            pltpu.VMEM((BLK,), jnp.float32),
            pltpu.VMEM((BLK, D), jnp.float32),
            pltpu.VMEM((npad,), jnp.float32),
            pltpu.VMEM((npad,), jnp.float32),
            pltpu.VMEM_SHARED((npad, D), jnp.float32),
            pltpu.SemaphoreType.DMA,
        ],
    )
    def k2(z_hbm, src_hbm, dst_hbm, w_hbm, s0_hbm, s1_hbm,
           out0_hbm, out1_hbm,
           src_v, dst_v, w_v, al_v, zr_v, s_v, tmp_v, out_sh, sem):
        cid = lax.axis_index("c")
        sid = lax.axis_index("s")
        wid = cid * NSUB + sid

        pltpu.sync_copy(s0_hbm, s_v)
        pltpu.sync_copy(s1_hbm, tmp_v)

        def sbody(i, c):
            ix = pl.ds(i * 16, 16)
            s_v[ix] = s_v[ix] + tmp_v[ix]
            return c
        lax.fori_loop(0, npad // 16, sbody, 0)

        zero16 = jnp.zeros((16,), jnp.float32)
        def zbody(i, c):
            r = i // 8
            cchunk = i % 8
            zr_v[r, pl.ds(cchunk * 16, 16)] = zero16
            return c
        lax.fori_loop(0, rows_per_copy * 8, zbody, 0)
        for j in range(sl // rows_per_copy):
            pltpu.sync_copy(
                zr_v, out_sh.at[pl.ds(sid * sl + j * rows_per_copy,
                                      rows_per_copy)])
        plsc.subcore_barrier()

        def block_body(i, c):
            b = wid + i * NW

            @pl.when(b < nblk)
            def _():
                base = b * BLK
                pltpu.sync_copy(src_hbm.at[pl.ds(base, BLK)], src_v)
                pltpu.sync_copy(dst_hbm.at[pl.ds(base, BLK)], dst_v)
                pltpu.sync_copy(w_hbm.at[pl.ds(base, BLK)], w_v)
                cp = pltpu.async_copy(z_hbm.at[src_v], zr_v, sem)
                for g in range(BLK // 16):
                    ix = pl.ds(g * 16, 16)
                    dv = dst_v[ix]
                    sg = plsc.load_gather(s_v, [dv])
                    al_v[ix] = w_v[ix] / sg
                cp.wait()

                def ebody(ei, c2):
                    ab = plsc.load_gather(
                        al_v, [jnp.full((16,), ei, jnp.int32)])
                    for cchunk in range(8):
                        ix = pl.ds(cchunk * 16, 16)
                        zr_v[ei, ix] = zr_v[ei, ix] * ab
                    return c2

                lax.fori_loop(0, BLK, ebody, 0)
                pltpu.sync_copy(zr_v, out_sh.at[dst_v], add=True)
            return c

        lax.fori_loop(0, iters, block_body, 0)
        plsc.subcore_barrier()

        for j in range(sl // rows_per_copy):
            r0 = sid * sl + j * rows_per_copy
            pltpu.sync_copy(out_sh.at[pl.ds(r0, rows_per_copy)], zr_v)

            @pl.when(cid == 0)
            def _():
                pltpu.sync_copy(zr_v, out0_hbm.at[pl.ds(r0, rows_per_copy)])

            @pl.when(cid == 1)
            def _():
                pltpu.sync_copy(zr_v, out1_hbm.at[pl.ds(r0, rows_per_copy)])

    return k2


def _add_body(a_ref, b_ref, o_ref):
    o_ref[...] = a_ref[...] + b_ref[...]


def _final_add(a, b, n):
    rb = 80
    grid = n // rb
    return pl.pallas_call(
        _add_body,
        grid=(grid,),
        in_specs=[
            pl.BlockSpec((rb, D), lambda i: (i, 0)),
            pl.BlockSpec((rb, D), lambda i: (i, 0)),
        ],
        out_specs=pl.BlockSpec((rb, D), lambda i: (i, 0)),
        out_shape=jax.ShapeDtypeStruct((n, D), jnp.float32),
    )(a, b)


def kernel(h, edge_index, W, beta):
    n = h.shape[0]
    e = edge_index.shape[1]
    npad = ((n + 2047) // 2048) * 2048

    z, zh = _prep(h, W, beta)
    src = edge_index[0]
    dst = edge_index[1]
    w, s0, s1 = _make_k1(e, npad)(zh, src, dst)
    out0, out1 = _make_k2(e, npad)(z, src, dst, w, s0, s1)
    return _final_add(out0, out1, n)


# speedup vs baseline: 6.7059x; 1.7113x over previous
import functools

import jax
import jax.numpy as jnp
from jax import lax
from jax.experimental import pallas as pl
from jax.experimental.pallas import tpu as pltpu
from jax.experimental.pallas import tpu_sc as plsc

D = 128
BLK = 128
NW = 32
NSUB = 16


def _prep_body(beta_ref, h_ref, w_ref, z_ref, zh_ref):
    z = lax.dot_general(h_ref[...], w_ref[...], (((1,), (1,)), ((), ())),
                        preferred_element_type=jnp.float32)
    z_ref[...] = z
    nrm = jnp.sqrt(jnp.sum(z * z, axis=1, keepdims=True))
    scale = jnp.sqrt(beta_ref[0, 0]) / jnp.maximum(nrm, 1e-6)
    zh_ref[...] = z * scale


def _prep(h, W, beta):
    n = h.shape[0]
    rb = 1000
    grid = n // rb
    return pl.pallas_call(
        _prep_body,
        grid=(grid,),
        in_specs=[
            pl.BlockSpec((1, 1), lambda i: (0, 0)),
            pl.BlockSpec((rb, D), lambda i: (i, 0)),
            pl.BlockSpec((D, D), lambda i: (0, 0)),
        ],
        out_specs=[
            pl.BlockSpec((rb, D), lambda i: (i, 0)),
            pl.BlockSpec((rb, D), lambda i: (i, 0)),
        ],
        out_shape=[
            jax.ShapeDtypeStruct((n, D), jnp.float32),
            jax.ShapeDtypeStruct((n, D), jnp.float32),
        ],
    )(beta.reshape(1, 1), h, W)


def _gram_body(a_ref, b_ref, o_ref):
    o_ref[...] = lax.dot_general(a_ref[...], b_ref[...],
                                 (((1,), (1,)), ((), ())),
                                 preferred_element_type=jnp.float32)


def _gram(zh):
    n = zh.shape[0]
    bm, bn = 1000, 2048
    return pl.pallas_call(
        _gram_body,
        grid=(n // bm, pl.cdiv(n, bn)),
        in_specs=[
            pl.BlockSpec((bm, D), lambda i, j: (i, 0)),
            pl.BlockSpec((bn, D), lambda i, j: (j, 0)),
        ],
        out_specs=pl.BlockSpec((bm, bn), lambda i, j: (i, j)),
        out_shape=jax.ShapeDtypeStruct((n, n), jnp.float32),
    )(zh, zh)


def _make_k1(e, n, npad):
    nblk = e // BLK
    iters = pl.cdiv(nblk, NW)
    sl = npad // NSUB
    mesh = plsc.VectorSubcoreMesh(core_axis_name="c", subcore_axis_name="s", num_cores=2, num_subcores=16)

    @functools.partial(
        pl.kernel,
        mesh=mesh,
        compiler_params=pltpu.CompilerParams(needs_layout_passes=False),
        out_type=(
            jax.ShapeDtypeStruct((e,), jnp.float32),
            jax.ShapeDtypeStruct((npad,), jnp.float32),
            jax.ShapeDtypeStruct((npad,), jnp.float32),
        ),
        scratch_types=[
            pltpu.VMEM((BLK,), jnp.int32),
            pltpu.VMEM((BLK,), jnp.int32),
            pltpu.VMEM((BLK,), jnp.int32),
            pltpu.VMEM((BLK,), jnp.float32),
            pltpu.VMEM((BLK,), jnp.float32),
            pltpu.VMEM((sl,), jnp.float32),
            pltpu.VMEM_SHARED((npad,), jnp.float32),
            pltpu.SemaphoreType.DMA,
        ],
    )
    def k1(gflat_hbm, src_hbm, dst_hbm, w_hbm, s0_hbm, s1_hbm,
           src_v, dst_v, gidx_v, g_v, w_v, sbuf_v, s_sh, sem):
        cid = lax.axis_index("c")
        sid = lax.axis_index("s")
        wid = cid * NSUB + sid

        zero16 = jnp.zeros((16,), jnp.float32)
        def zbody(i, c):
            sbuf_v[pl.ds(i * 16, 16)] = zero16
            return c
        lax.fori_loop(0, sl // 16, zbody, 0)
        pltpu.sync_copy(sbuf_v, s_sh.at[pl.ds(sid * sl, sl)])
        plsc.subcore_barrier()

        nconst = jnp.full((16,), n, jnp.int32)

        def block_body(i, c):
            b = wid + i * NW

            @pl.when(b < nblk)
            def _():
                base = b * BLK
                pltpu.sync_copy(src_hbm.at[pl.ds(base, BLK)], src_v)
                pltpu.sync_copy(dst_hbm.at[pl.ds(base, BLK)], dst_v)
                for g in range(BLK // 16):
                    ix = pl.ds(g * 16, 16)
                    gidx_v[ix] = src_v[ix] * nconst + dst_v[ix]
                pltpu.async_copy(gflat_hbm.at[gidx_v], g_v, sem).wait()
                for g in range(BLK // 16):
                    ix = pl.ds(g * 16, 16)
                    w_v[ix] = jnp.exp(g_v[ix])
                pltpu.sync_copy(w_v, w_hbm.at[pl.ds(base, BLK)])
                pltpu.sync_copy(w_v, s_sh.at[dst_v], add=True)
            return c

        lax.fori_loop(0, iters, block_body, 0)
        plsc.subcore_barrier()

        pltpu.sync_copy(s_sh.at[pl.ds(sid * sl, sl)], sbuf_v)

        @pl.when(cid == 0)
        def _():
            pltpu.sync_copy(sbuf_v, s0_hbm.at[pl.ds(sid * sl, sl)])

        @pl.when(cid == 1)
        def _():
            pltpu.sync_copy(sbuf_v, s1_hbm.at[pl.ds(sid * sl, sl)])

    return k1


def _make_k2(e, npad):
    nblk = e // BLK
    iters = pl.cdiv(nblk, NW)
    sl = npad // NSUB
    rows_per_copy = 128
    mesh = plsc.VectorSubcoreMesh(core_axis_name="c", subcore_axis_name="s", num_cores=2, num_subcores=16)

    @functools.partial(
        pl.kernel,
        mesh=mesh,
        compiler_params=pltpu.CompilerParams(needs_layout_passes=False),
        out_type=(
            jax.ShapeDtypeStruct((npad, D), jnp.float32),
            jax.ShapeDtypeStruct((npad, D), jnp.float32),
        ),
        scratch_types=[
            pltpu.VMEM((BLK,), jnp.int32),
            pltpu.VMEM((BLK,), jnp.int32),
            pltpu.VMEM((BLK,), jnp.float32),
            pltpu.VMEM((BLK,), jnp.float32),
            pltpu.VMEM((BLK, D), jnp.float32),
            pltpu.VMEM((npad,), jnp.float32),
            pltpu.VMEM((npad,), jnp.float32),
            pltpu.VMEM_SHARED((npad, D), jnp.float32),
            pltpu.SemaphoreType.DMA,
        ],
    )
    def k2(z_hbm, src_hbm, dst_hbm, w_hbm, s0_hbm, s1_hbm,
           out0_hbm, out1_hbm,
           src_v, dst_v, w_v, al_v, zr_v, s_v, tmp_v, out_sh, sem):
        cid = lax.axis_index("c")
        sid = lax.axis_index("s")
        wid = cid * NSUB + sid

        pltpu.sync_copy(s0_hbm, s_v)
        pltpu.sync_copy(s1_hbm, tmp_v)

        def sbody(i, c):
            ix = pl.ds(i * 16, 16)
            s_v[ix] = s_v[ix] + tmp_v[ix]
            return c
        lax.fori_loop(0, npad // 16, sbody, 0)

        zero16 = jnp.zeros((16,), jnp.float32)
        def zbody(i, c):
            r = i // 8
            cchunk = i % 8
            zr_v[r, pl.ds(cchunk * 16, 16)] = zero16
            return c
        lax.fori_loop(0, rows_per_copy * 8, zbody, 0)
        for j in range(sl // rows_per_copy):
            pltpu.sync_copy(
                zr_v, out_sh.at[pl.ds(sid * sl + j * rows_per_copy,
                                      rows_per_copy)])
        plsc.subcore_barrier()

        def block_body(i, c):
            b = wid + i * NW

            @pl.when(b < nblk)
            def _():
                base = b * BLK
                pltpu.sync_copy(src_hbm.at[pl.ds(base, BLK)], src_v)
                pltpu.sync_copy(dst_hbm.at[pl.ds(base, BLK)], dst_v)
                pltpu.sync_copy(w_hbm.at[pl.ds(base, BLK)], w_v)
                cp = pltpu.async_copy(z_hbm.at[src_v], zr_v, sem)
                for g in range(BLK // 16):
                    ix = pl.ds(g * 16, 16)
                    dv = dst_v[ix]
                    sg = plsc.load_gather(s_v, [dv])
                    al_v[ix] = w_v[ix] / sg
                cp.wait()

                def ebody(ei, c2):
                    ab = plsc.load_gather(
                        al_v, [jnp.full((16,), ei, jnp.int32)])
                    for cchunk in range(8):
                        ix = pl.ds(cchunk * 16, 16)
                        zr_v[ei, ix] = zr_v[ei, ix] * ab
                    return c2

                lax.fori_loop(0, BLK, ebody, 0)
                pltpu.sync_copy(zr_v, out_sh.at[dst_v], add=True)
            return c

        lax.fori_loop(0, iters, block_body, 0)
        plsc.subcore_barrier()

        for j in range(sl // rows_per_copy):
            r0 = sid * sl + j * rows_per_copy
            pltpu.sync_copy(out_sh.at[pl.ds(r0, rows_per_copy)], zr_v)

            @pl.when(cid == 0)
            def _():
                pltpu.sync_copy(zr_v, out0_hbm.at[pl.ds(r0, rows_per_copy)])

            @pl.when(cid == 1)
            def _():
                pltpu.sync_copy(zr_v, out1_hbm.at[pl.ds(r0, rows_per_copy)])

    return k2


def _add_body(a_ref, b_ref, o_ref):
    o_ref[...] = a_ref[...] + b_ref[...]


def _final_add(a, b, n):
    rb = 80
    grid = n // rb
    return pl.pallas_call(
        _add_body,
        grid=(grid,),
        in_specs=[
            pl.BlockSpec((rb, D), lambda i: (i, 0)),
            pl.BlockSpec((rb, D), lambda i: (i, 0)),
        ],
        out_specs=pl.BlockSpec((rb, D), lambda i: (i, 0)),
        out_shape=jax.ShapeDtypeStruct((n, D), jnp.float32),
    )(a, b)


def kernel(h, edge_index, W, beta):
    n = h.shape[0]
    e = edge_index.shape[1]
    npad = ((n + 2047) // 2048) * 2048

    z, zh = _prep(h, W, beta)
    gflat = _gram(zh).reshape(n * n)
    src = edge_index[0]
    dst = edge_index[1]
    w, s0, s1 = _make_k1(e, n, npad)(gflat, src, dst)
    out0, out1 = _make_k2(e, npad)(z, src, dst, w, s0, s1)
    return _final_add(out0, out1, n)


# speedup vs baseline: 6.7632x; 1.0086x over previous
import functools

import jax
import jax.numpy as jnp
from jax import lax
from jax.experimental import pallas as pl
from jax.experimental.pallas import tpu as pltpu
from jax.experimental.pallas import tpu_sc as plsc

D = 128
BLK = 128
NW = 32
NSUB = 16


def _prep_body(beta_ref, h_ref, w_ref, z_ref, zh_ref):
    z = lax.dot_general(h_ref[...], w_ref[...], (((1,), (1,)), ((), ())),
                        preferred_element_type=jnp.float32)
    z_ref[...] = z
    nrm = jnp.sqrt(jnp.sum(z * z, axis=1, keepdims=True))
    scale = jnp.sqrt(beta_ref[0, 0]) / jnp.maximum(nrm, 1e-6)
    zh_ref[...] = (z * scale).astype(jnp.bfloat16)


def _prep(h, W, beta):
    n = h.shape[0]
    rb = 1000
    grid = n // rb
    return pl.pallas_call(
        _prep_body,
        grid=(grid,),
        in_specs=[
            pl.BlockSpec((1, 1), lambda i: (0, 0)),
            pl.BlockSpec((rb, D), lambda i: (i, 0)),
            pl.BlockSpec((D, D), lambda i: (0, 0)),
        ],
        out_specs=[
            pl.BlockSpec((rb, D), lambda i: (i, 0)),
            pl.BlockSpec((rb, D), lambda i: (i, 0)),
        ],
        out_shape=[
            jax.ShapeDtypeStruct((n, D), jnp.float32),
            jax.ShapeDtypeStruct((n, D), jnp.bfloat16),
        ],
    )(beta.reshape(1, 1), h, W)


def _gram_body(a_ref, b_ref, o_ref):
    o_ref[...] = lax.dot_general(a_ref[...], b_ref[...],
                                 (((1,), (1,)), ((), ())),
                                 preferred_element_type=jnp.float32)


def _gram(zh):
    n = zh.shape[0]
    bm, bn = 1000, 2048
    return pl.pallas_call(
        _gram_body,
        grid=(n // bm, pl.cdiv(n, bn)),
        in_specs=[
            pl.BlockSpec((bm, D), lambda i, j: (i, 0)),
            pl.BlockSpec((bn, D), lambda i, j: (j, 0)),
        ],
        out_specs=pl.BlockSpec((bm, bn), lambda i, j: (i, j)),
        out_shape=jax.ShapeDtypeStruct((n, n), jnp.float32),
    )(zh, zh)


def _make_k1(e, n, npad):
    nblk = e // BLK
    iters = pl.cdiv(nblk, NW)
    sl = npad // NSUB
    mesh = plsc.VectorSubcoreMesh(core_axis_name="c", subcore_axis_name="s", num_cores=2, num_subcores=16)

    @functools.partial(
        pl.kernel,
        mesh=mesh,
        compiler_params=pltpu.CompilerParams(needs_layout_passes=False),
        out_type=(
            jax.ShapeDtypeStruct((e,), jnp.float32),
            jax.ShapeDtypeStruct((npad,), jnp.float32),
            jax.ShapeDtypeStruct((npad,), jnp.float32),
        ),
        scratch_types=[
            pltpu.VMEM((BLK,), jnp.int32),
            pltpu.VMEM((BLK,), jnp.int32),
            pltpu.VMEM((BLK,), jnp.int32),
            pltpu.VMEM((BLK,), jnp.float32),
            pltpu.VMEM((BLK,), jnp.float32),
            pltpu.VMEM((sl,), jnp.float32),
            pltpu.VMEM_SHARED((npad,), jnp.float32),
            pltpu.SemaphoreType.DMA,
        ],
    )
    def k1(gflat_hbm, src_hbm, dst_hbm, w_hbm, s0_hbm, s1_hbm,
           src_v, dst_v, gidx_v, g_v, w_v, sbuf_v, s_sh, sem):
        cid = lax.axis_index("c")
        sid = lax.axis_index("s")
        wid = cid * NSUB + sid

        zero16 = jnp.zeros((16,), jnp.float32)
        def zbody(i, c):
            sbuf_v[pl.ds(i * 16, 16)] = zero16
            return c
        lax.fori_loop(0, sl // 16, zbody, 0)
        pltpu.sync_copy(sbuf_v, s_sh.at[pl.ds(sid * sl, sl)])
        plsc.subcore_barrier()

        nconst = jnp.full((16,), n, jnp.int32)

        def block_body(i, c):
            b = wid + i * NW

            @pl.when(b < nblk)
            def _():
                base = b * BLK
                pltpu.sync_copy(src_hbm.at[pl.ds(base, BLK)], src_v)
                pltpu.sync_copy(dst_hbm.at[pl.ds(base, BLK)], dst_v)
                for g in range(BLK // 16):
                    ix = pl.ds(g * 16, 16)
                    gidx_v[ix] = src_v[ix] * nconst + dst_v[ix]
                pltpu.async_copy(gflat_hbm.at[gidx_v], g_v, sem).wait()
                for g in range(BLK // 16):
                    ix = pl.ds(g * 16, 16)
                    w_v[ix] = jnp.exp(g_v[ix])
                pltpu.sync_copy(w_v, w_hbm.at[pl.ds(base, BLK)])
                pltpu.sync_copy(w_v, s_sh.at[dst_v], add=True)
            return c

        lax.fori_loop(0, iters, block_body, 0)
        plsc.subcore_barrier()

        pltpu.sync_copy(s_sh.at[pl.ds(sid * sl, sl)], sbuf_v)

        @pl.when(cid == 0)
        def _():
            pltpu.sync_copy(sbuf_v, s0_hbm.at[pl.ds(sid * sl, sl)])

        @pl.when(cid == 1)
        def _():
            pltpu.sync_copy(sbuf_v, s1_hbm.at[pl.ds(sid * sl, sl)])

    return k1


def _make_k2(e, npad):
    nblk = e // BLK
    iters = pl.cdiv(nblk, NW)
    sl = npad // NSUB
    rows_per_copy = 128
    mesh = plsc.VectorSubcoreMesh(core_axis_name="c", subcore_axis_name="s", num_cores=2, num_subcores=16)

    @functools.partial(
        pl.kernel,
        mesh=mesh,
        compiler_params=pltpu.CompilerParams(needs_layout_passes=False),
        out_type=(
            jax.ShapeDtypeStruct((npad, D), jnp.float32),
            jax.ShapeDtypeStruct((npad, D), jnp.float32),
        ),
        scratch_types=[
            pltpu.VMEM((BLK,), jnp.int32),
            pltpu.VMEM((BLK,), jnp.int32),
            pltpu.VMEM((BLK,), jnp.float32),
            pltpu.VMEM((BLK,), jnp.float32),
            pltpu.VMEM((BLK, D), jnp.float32),
            pltpu.VMEM((npad,), jnp.float32),
            pltpu.VMEM((npad,), jnp.float32),
            pltpu.VMEM_SHARED((npad, D), jnp.float32),
            pltpu.SemaphoreType.DMA,
        ],
    )
    def k2(z_hbm, src_hbm, dst_hbm, w_hbm, s0_hbm, s1_hbm,
           out0_hbm, out1_hbm,
           src_v, dst_v, w_v, al_v, zr_v, s_v, tmp_v, out_sh, sem):
        cid = lax.axis_index("c")
        sid = lax.axis_index("s")
        wid = cid * NSUB + sid

        pltpu.sync_copy(s0_hbm, s_v)
        pltpu.sync_copy(s1_hbm, tmp_v)

        def sbody(i, c):
            ix = pl.ds(i * 16, 16)
            s_v[ix] = s_v[ix] + tmp_v[ix]
            return c
        lax.fori_loop(0, npad // 16, sbody, 0)

        zero16 = jnp.zeros((16,), jnp.float32)
        def zbody(i, c):
            r = i // 8
            cchunk = i % 8
            zr_v[r, pl.ds(cchunk * 16, 16)] = zero16
            return c
        lax.fori_loop(0, rows_per_copy * 8, zbody, 0)
        for j in range(sl // rows_per_copy):
            pltpu.sync_copy(
                zr_v, out_sh.at[pl.ds(sid * sl + j * rows_per_copy,
                                      rows_per_copy)])
        plsc.subcore_barrier()

        def block_body(i, c):
            b = wid + i * NW

            @pl.when(b < nblk)
            def _():
                base = b * BLK
                pltpu.sync_copy(src_hbm.at[pl.ds(base, BLK)], src_v)
                pltpu.sync_copy(dst_hbm.at[pl.ds(base, BLK)], dst_v)
                pltpu.sync_copy(w_hbm.at[pl.ds(base, BLK)], w_v)
                cp = pltpu.async_copy(z_hbm.at[src_v], zr_v, sem)
                for g in range(BLK // 16):
                    ix = pl.ds(g * 16, 16)
                    dv = dst_v[ix]
                    sg = plsc.load_gather(s_v, [dv])
                    al_v[ix] = w_v[ix] / sg
                cp.wait()

                def ebody(ei, c2):
                    ab = plsc.load_gather(
                        al_v, [jnp.full((16,), ei, jnp.int32)])
                    for cchunk in range(8):
                        ix = pl.ds(cchunk * 16, 16)
                        zr_v[ei, ix] = zr_v[ei, ix] * ab
                    return c2

                lax.fori_loop(0, BLK, ebody, 0)
                pltpu.sync_copy(zr_v, out_sh.at[dst_v], add=True)
            return c

        lax.fori_loop(0, iters, block_body, 0)
        plsc.subcore_barrier()

        for j in range(sl // rows_per_copy):
            r0 = sid * sl + j * rows_per_copy
            pltpu.sync_copy(out_sh.at[pl.ds(r0, rows_per_copy)], zr_v)

            @pl.when(cid == 0)
            def _():
                pltpu.sync_copy(zr_v, out0_hbm.at[pl.ds(r0, rows_per_copy)])

            @pl.when(cid == 1)
            def _():
                pltpu.sync_copy(zr_v, out1_hbm.at[pl.ds(r0, rows_per_copy)])

    return k2


def _add_body(a_ref, b_ref, o_ref):
    o_ref[...] = a_ref[...] + b_ref[...]


def _final_add(a, b, n):
    rb = 80
    grid = n // rb
    return pl.pallas_call(
        _add_body,
        grid=(grid,),
        in_specs=[
            pl.BlockSpec((rb, D), lambda i: (i, 0)),
            pl.BlockSpec((rb, D), lambda i: (i, 0)),
        ],
        out_specs=pl.BlockSpec((rb, D), lambda i: (i, 0)),
        out_shape=jax.ShapeDtypeStruct((n, D), jnp.float32),
    )(a, b)


def kernel(h, edge_index, W, beta):
    n = h.shape[0]
    e = edge_index.shape[1]
    npad = ((n + 2047) // 2048) * 2048

    z, zh = _prep(h, W, beta)
    gflat = _gram(zh).reshape(n * n)
    src = edge_index[0]
    dst = edge_index[1]
    w, s0, s1 = _make_k1(e, n, npad)(gflat, src, dst)
    out0, out1 = _make_k2(e, npad)(z, src, dst, w, s0, s1)
    return _final_add(out0, out1, n)
